# baseline (speedup 1.0000x reference)
import numpy as np
import jax
import jax.numpy as jnp
from jax import lax
from jax.experimental import pallas as pl
from jax.experimental.pallas import tpu as pltpu

_F32 = jnp.float32
_TN = (((0,), (0,)), ((), ()))


def _row_softmax(l):
    m = jnp.max(l, axis=-1, keepdims=True)
    p = jnp.exp(l - m)
    return p / jnp.sum(p, axis=-1, keepdims=True)


def _leaky(x):
    return jnp.where(x >= 0, x, 0.05 * x)


def _first_stage(x2, sel, W, b):
    B, C, TV = x2.shape
    V = sel.shape[1]
    O = W.shape[0]

    def body(x_ref, sel_ref, w_ref, bcol_ref, h_ref, a_ref):
        xb = x_ref[0]
        e = jnp.dot(xb, sel_ref[...], preferred_element_type=_F32)
        l = lax.dot_general(e, e, _TN, preferred_element_type=_F32)
        a_ref[0] = _row_softmax(l * (1.0 / np.sqrt(C)))
        h = jnp.dot(w_ref[...], xb, preferred_element_type=_F32)
        h_ref[0] = h + bcol_ref[...]

    return pl.pallas_call(
        body,
        grid=(B,),
        in_specs=[
            pl.BlockSpec((1, C, TV), lambda b: (b, 0, 0)),
            pl.BlockSpec((TV, V), lambda b: (0, 0)),
            pl.BlockSpec((O, C), lambda b: (0, 0)),
            pl.BlockSpec((O, 1), lambda b: (0, 0)),
        ],
        out_specs=[
            pl.BlockSpec((1, O, TV), lambda b: (b, 0, 0)),
            pl.BlockSpec((1, V, V), lambda b: (b, 0, 0)),
        ],
        out_shape=[
            jax.ShapeDtypeStruct((B, O, TV), _F32),
            jax.ShapeDtypeStruct((B, V, V), _F32),
        ],
    )(x2, sel, W, b.reshape(O, 1))


def _amix(H, A, V, n_chunks):
    B, R, _ = H.shape
    Rc = R // n_chunks

    def body(h_ref, a_ref, y_ref):
        y_ref[0] = jnp.dot(h_ref[0], a_ref[0], preferred_element_type=_F32)

    return pl.pallas_call(
        body,
        grid=(B, n_chunks),
        in_specs=[
            pl.BlockSpec((1, Rc, V), lambda b, k: (b, k, 0)),
            pl.BlockSpec((1, V, V), lambda b, k: (b, 0, 0)),
        ],
        out_specs=pl.BlockSpec((1, Rc, V), lambda b, k: (b, k, 0)),
        out_shape=jax.ShapeDtypeStruct((B, R, V), _F32),
    )(H, A)


def _mid_stage(Y, sel, g, be, W, b, V):
    B, C, TV = Y.shape
    O = W.shape[0]
    n = B * TV

    def body(y_ref, sel_ref, g_ref, be_ref, w_ref, bcol_ref, h_ref, a_ref,
             acc):
        ph = pl.program_id(0)
        b_i = pl.program_id(1)

        @pl.when((ph == 0) & (b_i == 0))
        def _():
            acc[...] = jnp.zeros_like(acc)

        @pl.when(ph == 0)
        def _():
            y = y_ref[0]
            acc[:, 0:1] += jnp.sum(y, axis=1, keepdims=True)
            acc[:, 1:2] += jnp.sum(y * y, axis=1, keepdims=True)

        @pl.when(ph == 1)
        def _():
            mean = acc[:, 0:1] * (1.0 / n)
            var = acc[:, 1:2] * (1.0 / n) - mean * mean
            inv = lax.rsqrt(var + 1e-5)
            scale = g_ref[...] * inv
            shift = be_ref[...] - mean * scale
            z = _leaky(y_ref[0] * scale + shift)
            h = jnp.dot(w_ref[...], z, preferred_element_type=_F32)
            h_ref[0] = h + bcol_ref[...]
            e = jnp.dot(z, sel_ref[...], preferred_element_type=_F32)
            l = lax.dot_general(e, e, _TN, preferred_element_type=_F32)
            a_ref[0] = _row_softmax(l * (1.0 / np.sqrt(C)))

    return pl.pallas_call(
        body,
        grid=(2, B),
        in_specs=[
            pl.BlockSpec((1, C, TV), lambda ph, b: (b, 0, 0)),
            pl.BlockSpec((TV, V), lambda ph, b: (0, 0)),
            pl.BlockSpec((C, 1), lambda ph, b: (0, 0)),
            pl.BlockSpec((C, 1), lambda ph, b: (0, 0)),
            pl.BlockSpec((O, C), lambda ph, b: (0, 0)),
            pl.BlockSpec((O, 1), lambda ph, b: (0, 0)),
        ],
        out_specs=[
            pl.BlockSpec((1, O, TV), lambda ph, b: (b, 0, 0)),
            pl.BlockSpec((1, V, V), lambda ph, b: (b, 0, 0)),
        ],
        out_shape=[
            jax.ShapeDtypeStruct((B, O, TV), _F32),
            jax.ShapeDtypeStruct((B, V, V), _F32),
        ],
        scratch_shapes=[pltpu.VMEM((C, 8), _F32)],
    )(Y, sel, g.reshape(C, 1), be.reshape(C, 1), W, b.reshape(O, 1))


def kernel(x, W1, b1, g1, be1, W2, b2, g2, be2, W3, b3):
    B, C0, T, V = x.shape
    O1, O2, O3 = W1.shape[0], W2.shape[0], W3.shape[0]
    TV = T * V
    x2 = x.reshape(B, C0, TV)
    sel = jnp.tile(jnp.eye(V, dtype=_F32) * (1.0 / T), (T, 1))

    H1, A1 = _first_stage(x2, sel, W1, b1)
    Y1 = _amix(H1.reshape(B, O1 * T, V), A1, V, 4)
    H2, A2 = _mid_stage(Y1.reshape(B, O1, TV), sel, g1, be1, W2, b2, V)
    Y2 = _amix(H2.reshape(B, O2 * T, V), A2, V, 2)
    H3, A3 = _mid_stage(Y2.reshape(B, O2, TV), sel, g2, be2, W3, b3, V)
    Y3 = _amix(H3.reshape(B, O3 * T, V), A3, V, 1)
    return Y3.reshape(B, O3, T, V)


# speedup vs baseline: 1.9747x; 1.9747x over previous
import numpy as np
import jax
import jax.numpy as jnp
from jax import lax
from jax.experimental import pallas as pl
from jax.experimental.pallas import tpu as pltpu

_F32 = jnp.float32
_TN = (((0,), (0,)), ((), ()))
_NT = (((1,), (1,)), ((), ()))
_PK = 5


def _row_softmax(l):
    m = jnp.max(l, axis=-1, keepdims=True)
    p = jnp.exp(l - m)
    return p / jnp.sum(p, axis=-1, keepdims=True)


def _leaky(x):
    return jnp.where(x >= 0, x, 0.05 * x)


def _consts(T, V):
    G = T // _PK
    VP = _PK * V
    sel_flat = np.tile(np.eye(V, dtype=np.float32) / T, (T, 1))
    blk = np.zeros((128, V), dtype=np.float32)
    for l in range(VP):
        blk[l, l % V] = 1.0 / T
    sel_pack = np.tile(blk, (G, 1))
    spread = (blk > 0).astype(np.float32)
    li = np.arange(128)
    kmask = ((li[:, None] // V) == (li[None, :] // V)).astype(np.float32)
    kmask[VP:, :] = 0.0
    kmask[:, VP:] = 0.0
    return (jnp.asarray(sel_flat), jnp.asarray(sel_pack), jnp.asarray(spread),
            jnp.asarray(kmask))


def _first_stage(x2, sel_flat, W, b, G):
    B, C, TV = x2.shape
    V = sel_flat.shape[1]
    VP = _PK * V
    O = W.shape[0]

    def body(x_ref, sel_ref, w_ref, bcol_ref, h_ref, a_ref):
        xb = x_ref[0]
        e = jnp.dot(xb, sel_ref[...], preferred_element_type=_F32)
        l = lax.dot_general(e, e, _TN, preferred_element_type=_F32)
        a_ref[0] = _row_softmax(l * (1.0 / np.sqrt(C)))
        h = jnp.dot(w_ref[...], xb, preferred_element_type=_F32)
        h = h + bcol_ref[...]
        zpad = jnp.zeros((O, 128 - VP), _F32)
        for g in range(G):
            h_ref[0, :, g * 128:g * 128 + VP] = h[:, g * VP:(g + 1) * VP]
            h_ref[0, :, g * 128 + VP:(g + 1) * 128] = zpad

    return pl.pallas_call(
        body,
        grid=(B,),
        in_specs=[
            pl.BlockSpec((1, C, TV), lambda b: (b, 0, 0)),
            pl.BlockSpec((TV, V), lambda b: (0, 0)),
            pl.BlockSpec((O, C), lambda b: (0, 0)),
            pl.BlockSpec((O, 1), lambda b: (0, 0)),
        ],
        out_specs=[
            pl.BlockSpec((1, O, G * 128), lambda b: (b, 0, 0)),
            pl.BlockSpec((1, V, V), lambda b: (b, 0, 0)),
        ],
        out_shape=[
            jax.ShapeDtypeStruct((B, O, G * 128), _F32),
            jax.ShapeDtypeStruct((B, V, V), _F32),
        ],
    )(x2, sel_flat, W, b.reshape(O, 1))


def _amix(H, A, spread, kmask):
    B, R, _ = H.shape

    def body(h_ref, a_ref, s_ref, km_ref, y_ref):
        sa = jnp.dot(s_ref[...], a_ref[0], preferred_element_type=_F32)
        blka = lax.dot_general(sa, s_ref[...], _NT,
                               preferred_element_type=_F32) * km_ref[...]
        y_ref[0] = jnp.dot(h_ref[0], blka, preferred_element_type=_F32)

    return pl.pallas_call(
        body,
        grid=(B,),
        in_specs=[
            pl.BlockSpec((1, R, 128), lambda b: (b, 0, 0)),
            pl.BlockSpec((1, A.shape[1], A.shape[2]), lambda b: (b, 0, 0)),
            pl.BlockSpec(spread.shape, lambda b: (0, 0)),
            pl.BlockSpec(kmask.shape, lambda b: (0, 0)),
        ],
        out_specs=pl.BlockSpec((1, R, 128), lambda b: (b, 0, 0)),
        out_shape=jax.ShapeDtypeStruct((B, R, 128), _F32),
    )(H, A, spread, kmask)


def _mid_stage(Y, sel_pack, g, be, W, b):
    B, C, L = Y.shape
    V = sel_pack.shape[1]
    O = W.shape[0]
    TV = L // 128 * _PK * V
    n = Y.shape[0] * TV

    def body(y_ref, sel_ref, g_ref, be_ref, w_ref, bcol_ref, h_ref, a_ref,
             acc):
        ph = pl.program_id(0)
        b_i = pl.program_id(1)

        @pl.when((ph == 0) & (b_i == 0))
        def _():
            acc[...] = jnp.zeros_like(acc)

        @pl.when(ph == 0)
        def _():
            y = y_ref[0]
            acc[:, 0:1] += jnp.sum(y, axis=1, keepdims=True)
            acc[:, 1:2] += jnp.sum(y * y, axis=1, keepdims=True)

        @pl.when(ph == 1)
        def _():
            mean = acc[:, 0:1] * (1.0 / n)
            var = acc[:, 1:2] * (1.0 / n) - mean * mean
            inv = lax.rsqrt(var + 1e-5)
            scale = g_ref[...] * inv
            shift = be_ref[...] - mean * scale
            z = _leaky(y_ref[0] * scale + shift)
            h = jnp.dot(w_ref[...], z, preferred_element_type=_F32)
            h_ref[0] = h + bcol_ref[...]
            e = jnp.dot(z, sel_ref[...], preferred_element_type=_F32)
            l = lax.dot_general(e, e, _TN, preferred_element_type=_F32)
            a_ref[0] = _row_softmax(l * (1.0 / np.sqrt(C)))

    return pl.pallas_call(
        body,
        grid=(2, B),
        in_specs=[
            pl.BlockSpec((1, C, L), lambda ph, b: (b, 0, 0)),
            pl.BlockSpec((L, V), lambda ph, b: (0, 0)),
            pl.BlockSpec((C, 1), lambda ph, b: (0, 0)),
            pl.BlockSpec((C, 1), lambda ph, b: (0, 0)),
            pl.BlockSpec((O, C), lambda ph, b: (0, 0)),
            pl.BlockSpec((O, 1), lambda ph, b: (0, 0)),
        ],
        out_specs=[
            pl.BlockSpec((1, O, L), lambda ph, b: (b, 0, 0)),
            pl.BlockSpec((1, V, V), lambda ph, b: (b, 0, 0)),
        ],
        out_shape=[
            jax.ShapeDtypeStruct((B, O, L), _F32),
            jax.ShapeDtypeStruct((B, V, V), _F32),
        ],
        scratch_shapes=[pltpu.VMEM((C, 8), _F32)],
    )(Y, sel_pack, g.reshape(C, 1), be.reshape(C, 1), W, b.reshape(O, 1))


def _last_mix(H, A, spread, kmask, O, T, V):
    B, R, _ = H.shape
    VP = _PK * V

    def body(h_ref, a_ref, s_ref, km_ref, y_ref):
        sa = jnp.dot(s_ref[...], a_ref[0], preferred_element_type=_F32)
        blka = lax.dot_general(sa, s_ref[...], _NT,
                               preferred_element_type=_F32) * km_ref[...]
        y = jnp.dot(h_ref[0], blka, preferred_element_type=_F32)
        y_ref[0] = y[:, 0:VP]

    return pl.pallas_call(
        body,
        grid=(B,),
        in_specs=[
            pl.BlockSpec((1, R, 128), lambda b: (b, 0, 0)),
            pl.BlockSpec((1, V, V), lambda b: (b, 0, 0)),
            pl.BlockSpec(spread.shape, lambda b: (0, 0)),
            pl.BlockSpec(kmask.shape, lambda b: (0, 0)),
        ],
        out_specs=pl.BlockSpec((1, R, VP), lambda b: (b, 0, 0)),
        out_shape=jax.ShapeDtypeStruct((B, R, VP), _F32),
    )(H, A, spread, kmask)


def kernel(x, W1, b1, g1, be1, W2, b2, g2, be2, W3, b3):
    B, C0, T, V = x.shape
    O1, O2, O3 = W1.shape[0], W2.shape[0], W3.shape[0]
    TV = T * V
    G = T // _PK
    L = G * 128
    sel_flat, sel_pack, spread, kmask = _consts(T, V)

    x2 = x.reshape(B, C0, TV)
    H1, A1 = _first_stage(x2, sel_flat, W1, b1, G)
    Y1 = _amix(H1.reshape(B, O1 * G, 128), A1, spread, kmask)
    H2, A2 = _mid_stage(Y1.reshape(B, O1, L), sel_pack, g1, be1, W2, b2)
    Y2 = _amix(H2.reshape(B, O2 * G, 128), A2, spread, kmask)
    H3, A3 = _mid_stage(Y2.reshape(B, O2, L), sel_pack, g2, be2, W3, b3)
    Y3 = _last_mix(H3.reshape(B, O3 * G, 128), A3, spread, kmask, O3, T, V)
    return Y3.reshape(B, O3, T, V)


# speedup vs baseline: 3.3658x; 1.7045x over previous
import numpy as np
import jax
import jax.numpy as jnp
from jax import lax
from jax.experimental import pallas as pl
from jax.experimental.pallas import tpu as pltpu

_F32 = jnp.float32
_TN = (((0,), (0,)), ((), ()))
_NT = (((1,), (1,)), ((), ()))
_PK = 5


def _row_softmax(l):
    m = jnp.max(l, axis=-1, keepdims=True)
    p = jnp.exp(l - m)
    return p / jnp.sum(p, axis=-1, keepdims=True)


def _leaky(x):
    return jnp.where(x >= 0, x, 0.05 * x)


def _consts(T, V):
    G = T // _PK
    VP = _PK * V
    sel_flat = np.tile(np.eye(V, dtype=np.float32) / T, (T, 1))
    blk = np.zeros((128, V), dtype=np.float32)
    for l in range(VP):
        blk[l, l % V] = 1.0 / T
    sel_pack = np.tile(blk, (G, 1))
    spread = (blk > 0).astype(np.float32)
    li = np.arange(128)
    kmask = ((li[:, None] // V) == (li[None, :] // V)).astype(np.float32)
    kmask[VP:, :] = 0.0
    kmask[:, VP:] = 0.0
    return (jnp.asarray(sel_flat), jnp.asarray(sel_pack), jnp.asarray(spread),
            jnp.asarray(kmask))


def _block_a(a, s_ref, km_ref):
    sa = jnp.dot(s_ref[...], a, preferred_element_type=_F32)
    return lax.dot_general(sa, s_ref[...], _NT,
                           preferred_element_type=_F32) * km_ref[...]


def _layer1(x2, sel_flat, spread, kmask, W, b, G):
    B, C, TV = x2.shape
    V = sel_flat.shape[1]
    VP = _PK * V
    O = W.shape[0]
    L = G * 128

    def body(x_ref, sel_ref, s_ref, km_ref, w_ref, bcol_ref, y_ref, st_ref):
        b_i = pl.program_id(0)
        xb = x_ref[0]
        e = jnp.dot(xb, sel_ref[...], preferred_element_type=_F32)
        lg = lax.dot_general(e, e, _TN, preferred_element_type=_F32)
        blka = _block_a(_row_softmax(lg * (1.0 / np.sqrt(C))), s_ref, km_ref)
        h = jnp.dot(w_ref[...], xb, preferred_element_type=_F32)
        h = h + bcol_ref[...]
        zpad = jnp.zeros((O, 128 - VP), _F32)
        ssum = jnp.zeros((O, 1), _F32)
        ssq = jnp.zeros((O, 1), _F32)
        for g in range(G):
            hg = jnp.concatenate([h[:, g * VP:(g + 1) * VP], zpad], axis=1)
            yg = jnp.dot(hg, blka, preferred_element_type=_F32)
            y_ref[0, :, g * 128:(g + 1) * 128] = yg
            ssum += jnp.sum(yg, axis=1, keepdims=True)
            ssq += jnp.sum(yg * yg, axis=1, keepdims=True)

        @pl.when(b_i == 0)
        def _():
            st_ref[...] = jnp.zeros((O, 2), _F32)

        st_ref[:, 0:1] += ssum
        st_ref[:, 1:2] += ssq

    return pl.pallas_call(
        body,
        grid=(B,),
        in_specs=[
            pl.BlockSpec((1, C, TV), lambda b: (b, 0, 0)),
            pl.BlockSpec((TV, V), lambda b: (0, 0)),
            pl.BlockSpec(spread.shape, lambda b: (0, 0)),
            pl.BlockSpec(kmask.shape, lambda b: (0, 0)),
            pl.BlockSpec((O, C), lambda b: (0, 0)),
            pl.BlockSpec((O, 1), lambda b: (0, 0)),
        ],
        out_specs=[
            pl.BlockSpec((1, O, L), lambda b: (b, 0, 0)),
            pl.BlockSpec((O, 2), lambda b: (0, 0)),
        ],
        out_shape=[
            jax.ShapeDtypeStruct((B, O, L), _F32),
            jax.ShapeDtypeStruct((O, 2), _F32),
        ],
    )(x2, sel_flat, spread, kmask, W, b.reshape(O, 1))


def _layer_n(Y, st, sel_pack, spread, kmask, g, be, W, b, last, V):
    B, C, L = Y.shape
    G = L // 128
    VP = _PK * V
    O = W.shape[0]
    n = B * G * VP

    def body(y_ref, st_in_ref, sel_ref, s_ref, km_ref, g_ref, be_ref, w_ref,
             bcol_ref, y_out_ref, st_ref):
        b_i = pl.program_id(0)
        mean = st_in_ref[:, 0:1] * (1.0 / n)
        var = st_in_ref[:, 1:2] * (1.0 / n) - mean * mean
        inv = lax.rsqrt(var + 1e-5)
        scale = g_ref[...] * inv
        shift = be_ref[...] - mean * scale
        z = _leaky(y_ref[0] * scale + shift)
        e = jnp.dot(z, sel_ref[...], preferred_element_type=_F32)
        lg = lax.dot_general(e, e, _TN, preferred_element_type=_F32)
        blka = _block_a(_row_softmax(lg * (1.0 / np.sqrt(C))), s_ref, km_ref)
        h = jnp.dot(w_ref[...], z, preferred_element_type=_F32)
        h = h + bcol_ref[...]
        if last:
            for gi in range(G):
                yg = jnp.dot(h[:, gi * 128:(gi + 1) * 128], blka,
                             preferred_element_type=_F32)
                y_out_ref[0, :, gi * VP:(gi + 1) * VP] = yg[:, 0:VP]
        else:
            ssum = jnp.zeros((O, 1), _F32)
            ssq = jnp.zeros((O, 1), _F32)
            for gi in range(G):
                yg = jnp.dot(h[:, gi * 128:(gi + 1) * 128], blka,
                             preferred_element_type=_F32)
                y_out_ref[0, :, gi * 128:(gi + 1) * 128] = yg
                ssum += jnp.sum(yg, axis=1, keepdims=True)
                ssq += jnp.sum(yg * yg, axis=1, keepdims=True)

            @pl.when(b_i == 0)
            def _():
                st_ref[...] = jnp.zeros((O, 2), _F32)

            st_ref[:, 0:1] += ssum
            st_ref[:, 1:2] += ssq

    l_out = G * VP if last else L
    outs = [
        pl.BlockSpec((1, O, l_out), lambda b: (b, 0, 0)),
        pl.BlockSpec((O, 2), lambda b: (0, 0)),
    ]
    out_sh = [
        jax.ShapeDtypeStruct((B, O, l_out), _F32),
        jax.ShapeDtypeStruct((O, 2), _F32),
    ]

    return pl.pallas_call(
        body,
        grid=(B,),
        in_specs=[
            pl.BlockSpec((1, C, L), lambda b: (b, 0, 0)),
            pl.BlockSpec((C, 2), lambda b: (0, 0)),
            pl.BlockSpec((L, V), lambda b: (0, 0)),
            pl.BlockSpec(spread.shape, lambda b: (0, 0)),
            pl.BlockSpec(kmask.shape, lambda b: (0, 0)),
            pl.BlockSpec((C, 1), lambda b: (0, 0)),
            pl.BlockSpec((C, 1), lambda b: (0, 0)),
            pl.BlockSpec((O, C), lambda b: (0, 0)),
            pl.BlockSpec((O, 1), lambda b: (0, 0)),
        ],
        out_specs=outs,
        out_shape=out_sh,
    )(Y, st, sel_pack, spread, kmask, g.reshape(C, 1), be.reshape(C, 1), W,
      b.reshape(O, 1))


def kernel(x, W1, b1, g1, be1, W2, b2, g2, be2, W3, b3):
    B, C0, T, V = x.shape
    O3 = W3.shape[0]
    G = T // _PK
    sel_flat, sel_pack, spread, kmask = _consts(T, V)

    x2 = x.reshape(B, C0, T * V)
    Y1, S1 = _layer1(x2, sel_flat, spread, kmask, W1, b1, G)
    Y2, S2 = _layer_n(Y1, S1, sel_pack, spread, kmask, g1, be1, W2, b2,
                      False, V)
    Y3, _ = _layer_n(Y2, S2, sel_pack, spread, kmask, g2, be2, W3, b3,
                     True, V)
    return Y3.reshape(B, O3, T, V)


# speedup vs baseline: 3.8609x; 1.1471x over previous
import numpy as np
import jax
import jax.numpy as jnp
from jax import lax
from jax.experimental import pallas as pl
from jax.experimental.pallas import tpu as pltpu

_F32 = jnp.float32
_TN = (((0,), (0,)), ((), ()))
_NT = (((1,), (1,)), ((), ()))
_PK = 5


def _row_softmax(l):
    m = jnp.max(l, axis=-1, keepdims=True)
    p = jnp.exp(l - m)
    return p / jnp.sum(p, axis=-1, keepdims=True)


def _leaky(x):
    return jnp.where(x >= 0, x, 0.05 * x)


def _consts(T, V):
    G = T // _PK
    VP = _PK * V
    sel_flat = np.tile(np.eye(V, dtype=np.float32) / T, (T, 1))
    blk = np.zeros((128, V), dtype=np.float32)
    for l in range(VP):
        blk[l, l % V] = 1.0 / T
    sel_pack = np.tile(blk, (G, 1))
    spread = (blk > 0).astype(np.float32)
    li = np.arange(128)
    kmask = ((li[:, None] // V) == (li[None, :] // V)).astype(np.float32)
    kmask[VP:, :] = 0.0
    kmask[:, VP:] = 0.0
    return (jnp.asarray(sel_flat), jnp.asarray(sel_pack), jnp.asarray(spread),
            jnp.asarray(kmask))


def _block_a(a, s_ref, km_ref):
    sa = jnp.dot(s_ref[...], a, preferred_element_type=_F32)
    return lax.dot_general(sa, s_ref[...], _NT,
                           preferred_element_type=_F32) * km_ref[...]


def _layer1(x2, sel_flat, spread, kmask, W, b, G):
    B, C, TV = x2.shape
    V = sel_flat.shape[1]
    VP = _PK * V
    O = W.shape[0]
    L = G * 128

    def body(x_ref, sel_ref, s_ref, km_ref, w_ref, bcol_ref, y_ref, st_ref):
        b_i = pl.program_id(0)
        xb = x_ref[0]
        e = jnp.dot(xb, sel_ref[...], preferred_element_type=_F32)
        lg = lax.dot_general(e, e, _TN, preferred_element_type=_F32)
        blka = _block_a(_row_softmax(lg * (1.0 / np.sqrt(C))), s_ref, km_ref)
        h = jnp.dot(w_ref[...], xb, preferred_element_type=_F32)
        h = h + bcol_ref[...]
        blka_v = blka[0:VP, :]
        sacc = jnp.zeros((O, 128), _F32)
        qacc = jnp.zeros((O, 128), _F32)
        for g in range(G):
            yg = jnp.dot(h[:, g * VP:(g + 1) * VP], blka_v,
                         preferred_element_type=_F32)
            y_ref[0, :, g * 128:(g + 1) * 128] = yg
            sacc += yg
            qacc += yg * yg

        @pl.when(b_i == 0)
        def _():
            st_ref[...] = jnp.zeros((O, 2), _F32)

        st_ref[:, 0:1] += jnp.sum(sacc, axis=1, keepdims=True)
        st_ref[:, 1:2] += jnp.sum(qacc, axis=1, keepdims=True)

    return pl.pallas_call(
        body,
        grid=(B,),
        in_specs=[
            pl.BlockSpec((1, C, TV), lambda b: (b, 0, 0)),
            pl.BlockSpec((TV, V), lambda b: (0, 0)),
            pl.BlockSpec(spread.shape, lambda b: (0, 0)),
            pl.BlockSpec(kmask.shape, lambda b: (0, 0)),
            pl.BlockSpec((O, C), lambda b: (0, 0)),
            pl.BlockSpec((O, 1), lambda b: (0, 0)),
        ],
        out_specs=[
            pl.BlockSpec((1, O, L), lambda b: (b, 0, 0)),
            pl.BlockSpec((O, 2), lambda b: (0, 0)),
        ],
        out_shape=[
            jax.ShapeDtypeStruct((B, O, L), _F32),
            jax.ShapeDtypeStruct((O, 2), _F32),
        ],
    )(x2, sel_flat, spread, kmask, W, b.reshape(O, 1))


def _layer_n(Y, st, sel_pack, spread, kmask, g, be, W, b, last, V):
    B, C, L = Y.shape
    G = L // 128
    VP = _PK * V
    O = W.shape[0]
    n = B * G * VP

    def body(y_ref, st_in_ref, sel_ref, s_ref, km_ref, g_ref, be_ref, w_ref,
             bcol_ref, y_out_ref, st_ref):
        b_i = pl.program_id(0)
        mean = st_in_ref[:, 0:1] * (1.0 / n)
        var = st_in_ref[:, 1:2] * (1.0 / n) - mean * mean
        inv = lax.rsqrt(var + 1e-5)
        scale = g_ref[...] * inv
        shift = be_ref[...] - mean * scale
        z = _leaky(y_ref[0] * scale + shift)
        e = jnp.dot(z, sel_ref[...], preferred_element_type=_F32)
        lg = lax.dot_general(e, e, _TN, preferred_element_type=_F32)
        blka = _block_a(_row_softmax(lg * (1.0 / np.sqrt(C))), s_ref, km_ref)
        h = jnp.dot(w_ref[...], z, preferred_element_type=_F32)
        h = h + bcol_ref[...]
        if last:
            for gi in range(G):
                yg = jnp.dot(h[:, gi * 128:(gi + 1) * 128], blka,
                             preferred_element_type=_F32)
                y_out_ref[0, :, gi * VP:(gi + 1) * VP] = yg[:, 0:VP]
        else:
            sacc = jnp.zeros((O, 128), _F32)
            qacc = jnp.zeros((O, 128), _F32)
            for gi in range(G):
                yg = jnp.dot(h[:, gi * 128:(gi + 1) * 128], blka,
                             preferred_element_type=_F32)
                y_out_ref[0, :, gi * 128:(gi + 1) * 128] = yg
                sacc += yg
                qacc += yg * yg

            @pl.when(b_i == 0)
            def _():
                st_ref[...] = jnp.zeros((O, 2), _F32)

            st_ref[:, 0:1] += jnp.sum(sacc, axis=1, keepdims=True)
            st_ref[:, 1:2] += jnp.sum(qacc, axis=1, keepdims=True)

    l_out = G * VP if last else L
    outs = [
        pl.BlockSpec((1, O, l_out), lambda b: (b, 0, 0)),
        pl.BlockSpec((O, 2), lambda b: (0, 0)),
    ]
    out_sh = [
        jax.ShapeDtypeStruct((B, O, l_out), _F32),
        jax.ShapeDtypeStruct((O, 2), _F32),
    ]

    return pl.pallas_call(
        body,
        grid=(B,),
        in_specs=[
            pl.BlockSpec((1, C, L), lambda b: (b, 0, 0)),
            pl.BlockSpec((C, 2), lambda b: (0, 0)),
            pl.BlockSpec((L, V), lambda b: (0, 0)),
            pl.BlockSpec(spread.shape, lambda b: (0, 0)),
            pl.BlockSpec(kmask.shape, lambda b: (0, 0)),
            pl.BlockSpec((C, 1), lambda b: (0, 0)),
            pl.BlockSpec((C, 1), lambda b: (0, 0)),
            pl.BlockSpec((O, C), lambda b: (0, 0)),
            pl.BlockSpec((O, 1), lambda b: (0, 0)),
        ],
        out_specs=outs,
        out_shape=out_sh,
    )(Y, st, sel_pack, spread, kmask, g.reshape(C, 1), be.reshape(C, 1), W,
      b.reshape(O, 1))


def kernel(x, W1, b1, g1, be1, W2, b2, g2, be2, W3, b3):
    B, C0, T, V = x.shape
    O3 = W3.shape[0]
    G = T // _PK
    sel_flat, sel_pack, spread, kmask = _consts(T, V)

    x2 = x.reshape(B, C0, T * V)
    Y1, S1 = _layer1(x2, sel_flat, spread, kmask, W1, b1, G)
    Y2, S2 = _layer_n(Y1, S1, sel_pack, spread, kmask, g1, be1, W2, b2,
                      False, V)
    Y3, _ = _layer_n(Y2, S2, sel_pack, spread, kmask, g2, be2, W3, b3,
                     True, V)
    return Y3.reshape(B, O3, T, V)
